# Initial kernel scaffold; baseline (speedup 1.0000x reference)
import jax
import jax.numpy as jnp
from jax.experimental import pallas as pl


def kernel(x, edge_index, edge_types, eps, params):
    raise NotImplementedError("write your pallas kernel here")


# speedup vs baseline: 23.4308x; 23.4308x over previous
import functools

import jax
import jax.numpy as jnp
from jax import lax
from jax.experimental import pallas as pl
from jax.experimental.pallas import tpu as pltpu
from jax.experimental.pallas import tpu_sc as plsc

N_NODES = 10000
N_EDGES = 320000
R = 4

_NC = 2
_NS = 16
_HALF = N_NODES // _NC
_ROWS = _HALF * R
_TRASH = 480
_ACC_ROWS = _ROWS + _TRASH
_EPT = N_EDGES // _NS
_CH = 80
_SUB = 25
_BLK = _SUB * _CH
_NBLK = _EPT // _BLK
_ZCH = _ACC_ROWS // _NS // _CH


@functools.lru_cache(maxsize=None)
def _make_sc_scatter(D):
    mesh = plsc.VectorSubcoreMesh(core_axis_name="c", subcore_axis_name="s",
                                  num_cores=_NC, num_subcores=_NS)

    @functools.partial(
        pl.kernel,
        out_type=jax.ShapeDtypeStruct((_NC * _ROWS, D), jnp.float32),
        mesh=mesh,
        scratch_types=[
            pltpu.VMEM((_BLK,), jnp.int32),
            pltpu.VMEM((_BLK,), jnp.int32),
            pltpu.VMEM((1, _CH), jnp.int32),
            pltpu.VMEM((_CH, D), jnp.float32),
            pltpu.VMEM((_CH, D), jnp.float32),
            pltpu.VMEM_SHARED((_ACC_ROWS, D), jnp.float32),
            pltpu.SemaphoreType.DMA,
        ],
        compiler_params=pltpu.CompilerParams(use_tc_tiling_on_sc=False),
    )
    def body(h_hbm, src_hbm, gidx_hbm, out_hbm,
             srcv, gidxv, idxv, rowsv, zbuf, acc, sem):
        cid = lax.axis_index("c")
        sid = lax.axis_index("s")
        glo = cid * _ROWS

        zvec = jnp.zeros((16,), jnp.float32)
        for r in range(_CH):
            for c in range(D // 16):
                zbuf[r, pl.ds(c * 16, 16)] = zvec
        zb = sid * (_ACC_ROWS // _NS)
        for k in range(_ZCH):
            pltpu.sync_copy(zbuf, acc.at[pl.ds(zb + k * _CH, _CH)])
        plsc.subcore_barrier()

        lane = lax.iota(jnp.int32, 16)

        def block(bi, _):
            pltpu.sync_copy(src_hbm.at[sid, bi], srcv)
            pltpu.sync_copy(gidx_hbm.at[sid, bi], gidxv)

            def chunk(ci, _):
                eb = ci * _CH
                gather = pltpu.async_copy(
                    h_hbm.at[srcv.at[pl.ds(eb, _CH)]], rowsv, sem)
                for j in range(_CH // 16):
                    g = gidxv[pl.ds(eb + j * 16, 16)]
                    owned = (g >= glo) & (g < glo + _ROWS)
                    tr = (_ROWS + lax.rem(ci * (_CH // 16) + j,
                                          _TRASH // 16) * 16 + lane)
                    idxv[0, pl.ds(j * 16, 16)] = jnp.where(owned, g - glo, tr)
                gather.wait()
                pltpu.sync_copy(rowsv, acc.at[idxv.at[0]], add=True)
                return 0

            lax.fori_loop(0, _SUB, chunk, 0)
            return 0

        lax.fori_loop(0, _NBLK, block, 0)
        plsc.subcore_barrier()

        opt = 1248
        pltpu.sync_copy(acc.at[pl.ds(sid * opt, opt)],
                        out_hbm.at[pl.ds(cid * _ROWS + sid * opt, opt)])
        tail = _ROWS - _NS * opt
        @pl.when(sid == _NS - 1)
        def _():
            pltpu.sync_copy(acc.at[pl.ds(_NS * opt, tail)],
                            out_hbm.at[pl.ds(cid * _ROWS + _NS * opt, tail)])

    return body


def _tc1_body(x_ref, s1_ref, c1_ref, v1_ref, sw_ref, b_ref, g_ref, be_ref,
              h1_ref):
    c1 = c1_ref[...]
    blocks = []
    for r in range(R):
        w = c1[r, 0] * v1_ref[0]
        for b in range(1, R):
            w = w + c1[r, b] * v1_ref[b]
        blocks.append(w)
    W = jnp.concatenate(blocks, axis=0)
    out = (jnp.dot(s1_ref[...], W, preferred_element_type=jnp.float32)
           + jnp.dot(x_ref[...], sw_ref[...], preferred_element_type=jnp.float32)
           + b_ref[...])
    out = jnp.where(out > 0, out, 0.1 * out)
    mu = jnp.mean(out, axis=-1, keepdims=True)
    var = jnp.mean((out - mu) ** 2, axis=-1, keepdims=True)
    h1_ref[...] = (out - mu) * lax.rsqrt(var + 1e-5) * g_ref[...] + be_ref[...]


def _tc2_body(s2_ref, h1_ref, eps_ref, c2_ref, v2_ref, sw_ref, b_ref, g_ref,
              be_ref, gw1_ref, gb1_ref, gw2_ref, gb2_ref, muw_ref, mub_ref,
              lvw_ref, lvb_ref, lpw_ref, lpb_ref, new_ref, neb_ref, ndw1_ref,
              ndb1_ref, ndw2_ref, ndb2_ref, edw1_ref, edb1_ref, edw2_ref,
              edb2_ref, enw1_ref, enb1_ref, enw2_ref, enb2_ref, stw1_ref,
              stb1_ref, stw2_ref, stb2_ref,
              h2_ref, mu_ref, lv_ref, z_ref, rn_ref, re_ref, pe_ref, ps_ref):
    c2 = c2_ref[...]
    blocks = []
    for r in range(R):
        w = c2[r, 0] * v2_ref[0]
        for b in range(1, R):
            w = w + c2[r, b] * v2_ref[b]
        blocks.append(w)
    W = jnp.concatenate(blocks, axis=0)
    out = (jnp.dot(s2_ref[...], W, preferred_element_type=jnp.float32)
           + jnp.dot(h1_ref[...], sw_ref[...], preferred_element_type=jnp.float32)
           + b_ref[...])
    out = jnp.where(out > 0, out, 0.1 * out)
    m = jnp.mean(out, axis=-1, keepdims=True)
    var = jnp.mean((out - m) ** 2, axis=-1, keepdims=True)
    h2 = (out - m) * lax.rsqrt(var + 1e-5) * g_ref[...] + be_ref[...]
    h2_ref[...] = h2

    gate = jnp.dot(jnp.maximum(
        jnp.dot(h2, gw1_ref[...], preferred_element_type=jnp.float32)
        + gb1_ref[...], 0.0), gw2_ref[...],
        preferred_element_type=jnp.float32) + gb2_ref[...]
    gate = gate - jnp.max(gate, axis=0, keepdims=True)
    eg = jnp.exp(gate)
    a = eg / jnp.sum(eg, axis=0, keepdims=True)
    gemb = jnp.sum(a * h2, axis=0, keepdims=True)

    mu = jnp.clip(jnp.dot(gemb, muw_ref[...], preferred_element_type=jnp.float32)
                  + mub_ref[...], -5.0, 5.0)
    lv = jnp.clip(jnp.dot(gemb, lvw_ref[...], preferred_element_type=jnp.float32)
                  + lvb_ref[...], -10.0, 10.0)
    std = jnp.clip(jnp.exp(0.5 * lv), 1e-6, 1e6)
    z = mu + eps_ref[...] * std
    mu_ref[...] = mu
    lv_ref[...] = lv
    z_ref[...] = z

    zp = jnp.maximum(jnp.dot(z, lpw_ref[...], preferred_element_type=jnp.float32)
                     + lpb_ref[...], 0.0)
    nep = jnp.maximum(jnp.dot(h2, new_ref[...], preferred_element_type=jnp.float32)
                      + neb_ref[...], 0.0)
    rn = jnp.maximum(jnp.dot(zp + nep, ndw1_ref[...],
                             preferred_element_type=jnp.float32)
                     + ndb1_ref[...], 0.0)
    rn_ref[...] = jnp.dot(rn, ndw2_ref[...],
                          preferred_element_type=jnp.float32) + ndb2_ref[...]
    ei = jnp.concatenate([zp, zp], axis=1)
    re = jnp.maximum(jnp.dot(ei, edw1_ref[...],
                             preferred_element_type=jnp.float32)
                     + edb1_ref[...], 0.0)
    re_ref[...] = jnp.dot(re, edw2_ref[...],
                          preferred_element_type=jnp.float32) + edb2_ref[...]
    pe = jnp.maximum(jnp.dot(z, enw1_ref[0:32, :],
                             preferred_element_type=jnp.float32)
                     + enb1_ref[...], 0.0)
    pe_ref[...] = jnp.dot(pe, enw2_ref[...],
                          preferred_element_type=jnp.float32) + enb2_ref[...]
    ps = jnp.maximum(jnp.dot(z, stw1_ref[0:32, :],
                             preferred_element_type=jnp.float32)
                     + stb1_ref[...], 0.0)
    ps_ref[...] = jnp.dot(ps, stw2_ref[...],
                          preferred_element_type=jnp.float32) + stb2_ref[...]


def _sc_scatter(D, h, src3, gidx3):
    return _make_sc_scatter(D)(h, src3, gidx3)


def _prep_body(dst_ref, et_ref, out_ref):
    out_ref[...] = dst_ref[...] * R + et_ref[...]


def kernel(x, edge_index, edge_types, eps, params):
    p = params
    src = edge_index[0].astype(jnp.int32)
    dst = edge_index[1].astype(jnp.int32)
    et = edge_types.astype(jnp.int32)
    src3 = src.reshape(_NS, _NBLK, _BLK)

    gidx = pl.pallas_call(
        _prep_body,
        out_shape=jax.ShapeDtypeStruct((N_EDGES // 128, 128), jnp.int32),
    )(dst.reshape(N_EDGES // 128, 128), et.reshape(N_EDGES // 128, 128))
    gidx3 = gidx.reshape(_NS, _NBLK, _BLK)

    row2 = lambda v: v.reshape(1, -1)

    xpad = jnp.pad(x, ((0, 0), (0, 12)))
    s1 = _sc_scatter(16, xpad, src3, gidx3).reshape(N_NODES, R * 16)
    v1pad = jnp.pad(p['V1'], ((0, 0), (0, 12), (0, 0)))

    h1 = pl.pallas_call(
        _tc1_body,
        grid=(10,),
        in_specs=[
            pl.BlockSpec((1000, 4), lambda i: (i, 0)),
            pl.BlockSpec((1000, 64), lambda i: (i, 0)),
            pl.BlockSpec((4, 4), lambda i: (0, 0)),
            pl.BlockSpec((4, 16, 64), lambda i: (0, 0, 0)),
            pl.BlockSpec((4, 64), lambda i: (0, 0)),
            pl.BlockSpec((1, 64), lambda i: (0, 0)),
            pl.BlockSpec((1, 64), lambda i: (0, 0)),
            pl.BlockSpec((1, 64), lambda i: (0, 0)),
        ],
        out_specs=pl.BlockSpec((1000, 64), lambda i: (i, 0)),
        out_shape=jax.ShapeDtypeStruct((N_NODES, 64), jnp.float32),
    )(x, s1, p['c1'], v1pad, p['s1'], row2(p['b1']), row2(p['g1']),
      row2(p['be1']))

    s2 = _sc_scatter(64, h1, src3, gidx3).reshape(N_NODES, R * 64)

    outs = pl.pallas_call(
        _tc2_body,
        out_shape=(
            jax.ShapeDtypeStruct((N_NODES, 128), jnp.float32),
            jax.ShapeDtypeStruct((1, 32), jnp.float32),
            jax.ShapeDtypeStruct((1, 32), jnp.float32),
            jax.ShapeDtypeStruct((1, 32), jnp.float32),
            jax.ShapeDtypeStruct((N_NODES, 4), jnp.float32),
            jax.ShapeDtypeStruct((1, 3), jnp.float32),
            jax.ShapeDtypeStruct((1, 2), jnp.float32),
            jax.ShapeDtypeStruct((1, 9), jnp.float32),
        ),
    )(s2, h1, eps, p['c2'], p['V2'], p['s2'], row2(p['b2']), row2(p['g2']),
      row2(p['be2']), p['gW1'], row2(p['gb1']), p['gW2'], row2(p['gb2']),
      p['muW'], row2(p['mub']), p['lvW'], row2(p['lvb']), p['lpW'],
      row2(p['lpb']), p['neW'], row2(p['neb']), p['ndW1'], row2(p['ndb1']),
      p['ndW2'], row2(p['ndb2']), p['edW1'], row2(p['edb1']), p['edW2'],
      row2(p['edb2']), p['enW1'], row2(p['enb1']), p['enW2'], row2(p['enb2']),
      p['stW1'], row2(p['stb1']), p['stW2'], row2(p['stb2']))

    h2, mu, lv, z, rn, re_row, pe, ps = outs
    re = jnp.broadcast_to(re_row, (N_EDGES, 3))
    return (mu, lv, z, h2, rn, re, pe, ps)
